# Initial kernel scaffold; baseline (speedup 1.0000x reference)
import jax
import jax.numpy as jnp
from jax.experimental import pallas as pl


def kernel(x, edge_index, kernel_idx, emb, num_frames, gamma1, beta1, W1, b1, W2, b2, W_emb, b_emb):
    raise NotImplementedError("write your pallas kernel here")


# speedup vs baseline: 3.6601x; 3.6601x over previous
import functools

import jax
import jax.numpy as jnp
from jax import lax
from jax.experimental import pallas as pl
from jax.experimental.pallas import tpu as pltpu
from jax.experimental.pallas import tpu_sc as plsc

_EPS = 1e-6


def _pre_body(x_ref, g_ref, b_ref, w_ref, z_ref):
    h = x_ref[...]
    mu = jnp.mean(h, axis=1, keepdims=True)
    d = h - mu
    var = jnp.mean(d * d, axis=1, keepdims=True)
    hn = d * lax.rsqrt(var + _EPS)
    hn = hn * g_ref[...] + b_ref[...]
    hs = hn * jax.nn.sigmoid(hn)
    z_ref[...] = jnp.dot(hs, w_ref[...], preferred_element_type=jnp.float32)


def _mid_body(y_ref, b1_ref, emb_ref, we_ref, be_ref, w2_ref, z_ref):
    c = b1_ref.shape[-1]
    h1 = y_ref[0] + y_ref[1] + b1_ref[...]
    mu = jnp.mean(h1, axis=1, keepdims=True)
    d = h1 - mu
    var = jnp.mean(d * d, axis=1, keepdims=True)
    hn = d * lax.rsqrt(var + _EPS)
    e = emb_ref[...]
    e = e * jax.nn.sigmoid(e)
    eo = jnp.dot(e, we_ref[...], preferred_element_type=jnp.float32) + be_ref[...]
    scale = eo[:, :c]
    shift = eo[:, c:]
    h = hn * (1.0 + scale) + shift
    h = h * jax.nn.sigmoid(h)
    z_ref[...] = jnp.dot(h, w2_ref[...], preferred_element_type=jnp.float32)


def _post_body(y_ref, b2_ref, x_ref, o_ref):
    o_ref[...] = y_ref[0] + y_ref[1] + b2_ref[...] + x_ref[...]


def _seg_sum(z2d, src, dst, kidx, n, c, k, e):
    NC, NS = 2, 16
    CH = 80
    EPW = e // (NC * NS)
    NITER = EPW // CH
    ZB = 128
    NP = -(-n // (NS * ZB)) * (NS * ZB)
    RPT = NP // NS
    assert EPW * NC * NS == e and NITER * CH == EPW and (RPT // ZB) * ZB == RPT

    mesh = plsc.VectorSubcoreMesh(
        core_axis_name="c", subcore_axis_name="s", num_cores=NC, num_subcores=NS)

    @functools.partial(
        pl.kernel,
        out_type=jax.ShapeDtypeStruct((NC, NP, c), jnp.float32),
        mesh=mesh,
        scratch_types=[
            pltpu.VMEM((ZB, c), jnp.float32),
            pltpu.VMEM((CH, c), jnp.float32),
            pltpu.VMEM((CH,), jnp.int32),
            pltpu.VMEM((CH,), jnp.int32),
            pltpu.VMEM((CH,), jnp.int32),
            pltpu.VMEM((CH,), jnp.int32),
            pltpu.VMEM_SHARED((NP, c), jnp.float32),
            pltpu.SemaphoreType.DMA,
        ],
    )
    def seg(z_hbm, src_hbm, dst_hbm, kidx_hbm, out_hbm,
            zbuf, rows, si, ki, gi, di, yacc, sem):
        cc = lax.axis_index("c")
        ss = lax.axis_index("s")
        zero16 = jnp.zeros((16,), jnp.float32)

        def zb_body(i, carry):
            for j in range(c // 16):
                zbuf[i, pl.ds(j * 16, 16)] = zero16
            return carry
        lax.fori_loop(0, ZB, zb_body, 0)

        row0 = ss * RPT
        for r in range(RPT // ZB):
            pltpu.sync_copy(zbuf, yacc.at[pl.ds(row0 + r * ZB, ZB)])
        plsc.subcore_barrier()

        ebase = (cc * NS + ss) * EPW

        def chunk_body(i, carry):
            off = pl.multiple_of(ebase + i * CH, 8)
            pltpu.sync_copy(src_hbm.at[pl.ds(off, CH)], si)
            pltpu.sync_copy(kidx_hbm.at[pl.ds(off, CH)], ki)
            pltpu.sync_copy(dst_hbm.at[pl.ds(off, CH)], di)
            for j in range(CH // 16):
                sl = pl.ds(j * 16, 16)
                gi[sl] = si[sl] * k + ki[sl]
            pltpu.async_copy(z_hbm.at[gi], rows, sem).wait()
            pltpu.sync_copy(rows, yacc.at[di], add=True)
            return carry
        lax.fori_loop(0, NITER, chunk_body, 0)

        plsc.subcore_barrier()
        pltpu.sync_copy(yacc.at[pl.ds(row0, RPT)],
                        out_hbm.at[cc, pl.ds(row0, RPT)])

    return seg(z2d, src, dst, kidx)


def kernel(x, edge_index, kernel_idx, emb, num_frames, gamma1, beta1,
           W1, b1, W2, b2, W_emb, b_emb):
    n, c = x.shape
    k = W1.shape[0]
    e = kernel_idx.shape[0]
    emb_d = emb.shape[1]
    src = edge_index[0]
    dst = edge_index[1]
    W1r = jnp.transpose(W1, (1, 0, 2)).reshape(c, k * c)
    W2r = jnp.transpose(W2, (1, 0, 2)).reshape(c, k * c)

    BN = 200
    G = n // BN
    f32 = jnp.float32

    z1 = pl.pallas_call(
        _pre_body,
        grid=(G,),
        in_specs=[
            pl.BlockSpec((BN, c), lambda i: (i, 0)),
            pl.BlockSpec((1, c), lambda i: (0, 0)),
            pl.BlockSpec((1, c), lambda i: (0, 0)),
            pl.BlockSpec((c, k * c), lambda i: (0, 0)),
        ],
        out_specs=pl.BlockSpec((BN, k * c), lambda i: (i, 0)),
        out_shape=jax.ShapeDtypeStruct((n, k * c), f32),
    )(x, gamma1.reshape(1, c), beta1.reshape(1, c), W1r)

    y1 = _seg_sum(z1.reshape(n * k, c), src, dst, kernel_idx, n, c, k, e)

    z2 = pl.pallas_call(
        _mid_body,
        grid=(G,),
        in_specs=[
            pl.BlockSpec((2, BN, c), lambda i: (0, i, 0)),
            pl.BlockSpec((1, c), lambda i: (0, 0)),
            pl.BlockSpec((BN, emb_d), lambda i: (i, 0)),
            pl.BlockSpec((emb_d, 2 * c), lambda i: (0, 0)),
            pl.BlockSpec((1, 2 * c), lambda i: (0, 0)),
            pl.BlockSpec((c, k * c), lambda i: (0, 0)),
        ],
        out_specs=pl.BlockSpec((BN, k * c), lambda i: (i, 0)),
        out_shape=jax.ShapeDtypeStruct((n, k * c), f32),
    )(y1, b1.reshape(1, c), emb, W_emb, b_emb.reshape(1, 2 * c), W2r)

    y2 = _seg_sum(z2.reshape(n * k, c), src, dst, kernel_idx, n, c, k, e)

    out = pl.pallas_call(
        _post_body,
        grid=(G,),
        in_specs=[
            pl.BlockSpec((2, BN, c), lambda i: (0, i, 0)),
            pl.BlockSpec((1, c), lambda i: (0, 0)),
            pl.BlockSpec((BN, c), lambda i: (i, 0)),
        ],
        out_specs=pl.BlockSpec((BN, c), lambda i: (i, 0)),
        out_shape=jax.ShapeDtypeStruct((n, c), f32),
    )(y2, b2.reshape(1, c), x)

    return out


# speedup vs baseline: 5.7349x; 1.5669x over previous
import functools

import jax
import jax.numpy as jnp
from jax import lax
from jax.experimental import pallas as pl
from jax.experimental.pallas import tpu as pltpu
from jax.experimental.pallas import tpu_sc as plsc

_EPS = 1e-6


def _pre_body(x_ref, g_ref, b_ref, w_ref, src_ref, kidx_ref, z_ref, gidx_ref,
              *, k):
    h = x_ref[...]
    mu = jnp.mean(h, axis=1, keepdims=True)
    d = h - mu
    var = jnp.mean(d * d, axis=1, keepdims=True)
    hn = d * lax.rsqrt(var + _EPS)
    hn = hn * g_ref[...] + b_ref[...]
    hs = hn * jax.nn.sigmoid(hn)
    z_ref[...] = jnp.dot(hs, w_ref[...], preferred_element_type=jnp.float32)
    gidx_ref[...] = src_ref[...] * k + kidx_ref[...]


def _mid_body(y_ref, b1_ref, emb_ref, we_ref, be_ref, w2_ref, z_ref):
    c = b1_ref.shape[-1]
    h1 = y_ref[0] + y_ref[1] + b1_ref[...]
    mu = jnp.mean(h1, axis=1, keepdims=True)
    d = h1 - mu
    var = jnp.mean(d * d, axis=1, keepdims=True)
    hn = d * lax.rsqrt(var + _EPS)
    e = emb_ref[...]
    e = e * jax.nn.sigmoid(e)
    eo = jnp.dot(e, we_ref[...], preferred_element_type=jnp.float32) + be_ref[...]
    scale = eo[:, :c]
    shift = eo[:, c:]
    h = hn * (1.0 + scale) + shift
    h = h * jax.nn.sigmoid(h)
    z_ref[...] = jnp.dot(h, w2_ref[...], preferred_element_type=jnp.float32)


def _post_body(y_ref, b2_ref, x_ref, o_ref):
    o_ref[...] = y_ref[0] + y_ref[1] + b2_ref[...] + x_ref[...]


@functools.lru_cache(maxsize=None)
def _make_seg(n, c, e):
    NC, NS = 2, 16
    CH = 80
    EPW = e // (NC * NS)
    NITER = EPW // CH
    ZB = 128
    NP = -(-n // (NS * ZB)) * (NS * ZB)
    RPT = NP // NS
    assert EPW * NC * NS == e and NITER * CH == EPW and (RPT // ZB) * ZB == RPT
    assert NITER % 2 == 1

    mesh = plsc.VectorSubcoreMesh(
        core_axis_name="c", subcore_axis_name="s", num_cores=NC, num_subcores=NS)

    @functools.partial(
        pl.kernel,
        out_type=jax.ShapeDtypeStruct((NC, NP, c), jnp.float32),
        mesh=mesh,
        scratch_types=[
            pltpu.VMEM((ZB, c), jnp.float32),
            [pltpu.VMEM((CH, c), jnp.float32)] * 2,
            [pltpu.VMEM((CH,), jnp.int32)] * 2,
            [pltpu.VMEM((CH,), jnp.int32)] * 2,
            pltpu.VMEM_SHARED((NP, c), jnp.float32),
            [pltpu.SemaphoreType.DMA] * 6,
        ],
    )
    def seg(gidx_hbm, dst_hbm, z_hbm, out_hbm, zbuf, rows, gib, dib, yacc,
            sems):
        sg = sems[0:2]
        sn = sems[2:4]
        sd = sems[4:6]
        cc = lax.axis_index("c")
        ss = lax.axis_index("s")
        t = cc * NS + ss
        ebase = t * EPW

        def i_issue(i, b):
            off = pl.multiple_of(ebase + i * CH, 8)
            pltpu.async_copy(gidx_hbm.at[pl.ds(off, CH)], gib[b], sn[b])
            pltpu.async_copy(dst_hbm.at[pl.ds(off, CH)], dib[b], sd[b])

        def i_wait(i, b):
            off = pl.multiple_of(ebase + i * CH, 8)
            pltpu.make_async_copy(gidx_hbm.at[pl.ds(off, CH)], gib[b],
                                  sn[b]).wait()
            pltpu.make_async_copy(dst_hbm.at[pl.ds(off, CH)], dib[b],
                                  sd[b]).wait()

        def g_issue(b):
            pltpu.async_copy(z_hbm.at[gib[b]], rows[b], sg[b])

        def g_wait(b):
            pltpu.make_async_copy(z_hbm.at[gib[b]], rows[b], sg[b]).wait()

        def s_do(b):
            pltpu.sync_copy(rows[b], yacc.at[dib[b]], add=True)

        i_issue(0, 0)
        i_issue(1, 1)

        zero16 = jnp.zeros((16,), jnp.float32)

        def zb_body(i, carry):
            for j in range(c // 16):
                zbuf[i, pl.ds(j * 16, 16)] = zero16
            return carry
        lax.fori_loop(0, ZB, zb_body, 0)

        row0 = ss * RPT
        for r in range(RPT // ZB):
            pltpu.sync_copy(zbuf, yacc.at[pl.ds(row0 + r * ZB, ZB)])
        plsc.subcore_barrier()

        i_wait(0, 0)
        g_issue(0)

        def one(i, b):
            bo = 1 - b
            i_wait(i + 1, bo)
            g_issue(bo)
            g_wait(b)
            s_do(b)
            inx = jnp.minimum(i + 2, NITER - 1)
            i_issue(inx, b)

        def body2(j, carry):
            i0 = j * 2
            one(i0, 0)
            one(i0 + 1, 1)
            return carry
        lax.fori_loop(0, NITER // 2, body2, 0)

        g_wait(0)
        s_do(0)
        i_wait(NITER - 1, 1)

        plsc.subcore_barrier()
        pltpu.sync_copy(yacc.at[pl.ds(row0, RPT)],
                        out_hbm.at[cc, pl.ds(row0, RPT)])

    return seg


def _seg_sum(z2d, gidx, dst, n, c, e):
    seg = _make_seg(n, c, e)
    return seg(gidx, dst, z2d)


def kernel(x, edge_index, kernel_idx, emb, num_frames, gamma1, beta1,
           W1, b1, W2, b2, W_emb, b_emb):
    n, c = x.shape
    k = W1.shape[0]
    e = kernel_idx.shape[0]
    emb_d = emb.shape[1]
    src = edge_index[0]
    dst = edge_index[1]
    W1r = jnp.transpose(W1, (1, 0, 2)).reshape(c, k * c)
    W2r = jnp.transpose(W2, (1, 0, 2)).reshape(c, k * c)

    BN = 200
    G = n // BN
    BE = e // G
    f32 = jnp.float32

    z1, gidx3 = pl.pallas_call(
        functools.partial(_pre_body, k=k),
        grid=(G,),
        in_specs=[
            pl.BlockSpec((BN, c), lambda i: (i, 0)),
            pl.BlockSpec((1, c), lambda i: (0, 0)),
            pl.BlockSpec((1, c), lambda i: (0, 0)),
            pl.BlockSpec((c, k * c), lambda i: (0, 0)),
            pl.BlockSpec((1, 1, BE), lambda i: (i, 0, 0)),
            pl.BlockSpec((1, 1, BE), lambda i: (i, 0, 0)),
        ],
        out_specs=[
            pl.BlockSpec((BN, k * c), lambda i: (i, 0)),
            pl.BlockSpec((1, 1, BE), lambda i: (i, 0, 0)),
        ],
        out_shape=[
            jax.ShapeDtypeStruct((n, k * c), f32),
            jax.ShapeDtypeStruct((G, 1, BE), jnp.int32),
        ],
    )(x, gamma1.reshape(1, c), beta1.reshape(1, c), W1r,
      src.reshape(G, 1, BE), kernel_idx.reshape(G, 1, BE))

    gidx = gidx3.reshape(e)
    y1 = _seg_sum(z1.reshape(n * k, c), gidx, dst, n, c, e)

    z2 = pl.pallas_call(
        _mid_body,
        grid=(G,),
        in_specs=[
            pl.BlockSpec((2, BN, c), lambda i: (0, i, 0)),
            pl.BlockSpec((1, c), lambda i: (0, 0)),
            pl.BlockSpec((BN, emb_d), lambda i: (i, 0)),
            pl.BlockSpec((emb_d, 2 * c), lambda i: (0, 0)),
            pl.BlockSpec((1, 2 * c), lambda i: (0, 0)),
            pl.BlockSpec((c, k * c), lambda i: (0, 0)),
        ],
        out_specs=pl.BlockSpec((BN, k * c), lambda i: (i, 0)),
        out_shape=jax.ShapeDtypeStruct((n, k * c), f32),
    )(y1, b1.reshape(1, c), emb, W_emb, b_emb.reshape(1, 2 * c), W2r)

    y2 = _seg_sum(z2.reshape(n * k, c), gidx, dst, n, c, e)

    out = pl.pallas_call(
        _post_body,
        grid=(G,),
        in_specs=[
            pl.BlockSpec((2, BN, c), lambda i: (0, i, 0)),
            pl.BlockSpec((1, c), lambda i: (0, 0)),
            pl.BlockSpec((BN, c), lambda i: (i, 0)),
        ],
        out_specs=pl.BlockSpec((BN, c), lambda i: (i, 0)),
        out_shape=jax.ShapeDtypeStruct((n, c), f32),
    )(y2, b2.reshape(1, c), x)

    return out


# speedup vs baseline: 5.7463x; 1.0020x over previous
import functools

import jax
import jax.numpy as jnp
from jax import lax
from jax.experimental import pallas as pl
from jax.experimental.pallas import tpu as pltpu
from jax.experimental.pallas import tpu_sc as plsc

_EPS = 1e-6


def _pre_body(x_ref, g_ref, b_ref, w_ref, src_ref, kidx_ref, z_ref, gidx_ref,
              *, k):
    h = x_ref[...]
    mu = jnp.mean(h, axis=1, keepdims=True)
    d = h - mu
    var = jnp.mean(d * d, axis=1, keepdims=True)
    hn = d * lax.rsqrt(var + _EPS)
    hn = hn * g_ref[...] + b_ref[...]
    hs = hn * jax.nn.sigmoid(hn)
    z_ref[...] = jnp.dot(hs.astype(jnp.bfloat16), w_ref[...],
                         preferred_element_type=jnp.float32)
    gidx_ref[...] = src_ref[...] * k + kidx_ref[...]


def _emb_body(emb_ref, we_ref, be_ref, eo_ref):
    e = emb_ref[...]
    e = e * jax.nn.sigmoid(e)
    eo_ref[...] = jnp.dot(e.astype(jnp.bfloat16), we_ref[...],
                          preferred_element_type=jnp.float32) + be_ref[...]


def _mid_body(y_ref, b1_ref, eo_ref, w2_ref, z_ref):
    c = b1_ref.shape[-1]
    h1 = y_ref[0] + y_ref[1] + b1_ref[...]
    mu = jnp.mean(h1, axis=1, keepdims=True)
    d = h1 - mu
    var = jnp.mean(d * d, axis=1, keepdims=True)
    hn = d * lax.rsqrt(var + _EPS)
    eo = eo_ref[...]
    scale = eo[:, :c]
    shift = eo[:, c:]
    h = hn * (1.0 + scale) + shift
    h = h * jax.nn.sigmoid(h)
    z_ref[...] = jnp.dot(h.astype(jnp.bfloat16), w2_ref[...],
                         preferred_element_type=jnp.float32)


def _post_body(y_ref, b2_ref, x_ref, o_ref):
    o_ref[...] = y_ref[0] + y_ref[1] + b2_ref[...] + x_ref[...]


@functools.lru_cache(maxsize=None)
def _make_seg(n, c, e):
    NC, NS = 2, 16
    CH = 80
    EPW = e // (NC * NS)
    NITER = EPW // CH
    ZB = 128
    NP = -(-n // (NS * ZB)) * (NS * ZB)
    RPT = NP // NS
    assert EPW * NC * NS == e and NITER * CH == EPW and (RPT // ZB) * ZB == RPT
    assert NITER % 2 == 1

    mesh = plsc.VectorSubcoreMesh(
        core_axis_name="c", subcore_axis_name="s", num_cores=NC, num_subcores=NS)

    @functools.partial(
        pl.kernel,
        out_type=jax.ShapeDtypeStruct((NC, NP, c), jnp.float32),
        mesh=mesh,
        scratch_types=[
            pltpu.VMEM((ZB, c), jnp.float32),
            [pltpu.VMEM((CH, c), jnp.float32)] * 2,
            [pltpu.VMEM((CH,), jnp.int32)] * 2,
            [pltpu.VMEM((CH,), jnp.int32)] * 2,
            pltpu.VMEM_SHARED((NP, c), jnp.float32),
            [pltpu.SemaphoreType.DMA] * 6,
        ],
    )
    def seg(gidx_hbm, dst_hbm, z_hbm, out_hbm, zbuf, rows, gib, dib, yacc,
            sems):
        sg = sems[0:2]
        sn = sems[2:4]
        sd = sems[4:6]
        cc = lax.axis_index("c")
        ss = lax.axis_index("s")
        t = cc * NS + ss
        ebase = t * EPW

        def i_issue(i, b):
            off = pl.multiple_of(ebase + i * CH, 8)
            pltpu.async_copy(gidx_hbm.at[pl.ds(off, CH)], gib[b], sn[b])
            pltpu.async_copy(dst_hbm.at[pl.ds(off, CH)], dib[b], sd[b])

        def i_wait(i, b):
            off = pl.multiple_of(ebase + i * CH, 8)
            pltpu.make_async_copy(gidx_hbm.at[pl.ds(off, CH)], gib[b],
                                  sn[b]).wait()
            pltpu.make_async_copy(dst_hbm.at[pl.ds(off, CH)], dib[b],
                                  sd[b]).wait()

        def g_issue(b):
            pltpu.async_copy(z_hbm.at[gib[b]], rows[b], sg[b])

        def g_wait(b):
            pltpu.make_async_copy(z_hbm.at[gib[b]], rows[b], sg[b]).wait()

        def s_do(b):
            pltpu.sync_copy(rows[b], yacc.at[dib[b]], add=True)

        i_issue(0, 0)
        i_issue(1, 1)

        zero16 = jnp.zeros((16,), jnp.float32)

        def zb_body(i, carry):
            for j in range(c // 16):
                zbuf[i, pl.ds(j * 16, 16)] = zero16
            return carry
        lax.fori_loop(0, ZB, zb_body, 0)

        row0 = ss * RPT
        for r in range(RPT // ZB):
            pltpu.sync_copy(zbuf, yacc.at[pl.ds(row0 + r * ZB, ZB)])
        plsc.subcore_barrier()

        i_wait(0, 0)
        g_issue(0)

        def one(i, b):
            bo = 1 - b
            i_wait(i + 1, bo)
            g_issue(bo)
            g_wait(b)
            s_do(b)
            inx = jnp.minimum(i + 2, NITER - 1)
            i_issue(inx, b)

        def body2(j, carry):
            i0 = j * 2
            one(i0, 0)
            one(i0 + 1, 1)
            return carry
        lax.fori_loop(0, NITER // 2, body2, 0)

        g_wait(0)
        s_do(0)
        i_wait(NITER - 1, 1)

        plsc.subcore_barrier()
        pltpu.sync_copy(yacc.at[pl.ds(row0, RPT)],
                        out_hbm.at[cc, pl.ds(row0, RPT)])

    return seg


def _seg_sum(z2d, gidx, dst, n, c, e):
    seg = _make_seg(n, c, e)
    return seg(gidx, dst, z2d)


def kernel(x, edge_index, kernel_idx, emb, num_frames, gamma1, beta1,
           W1, b1, W2, b2, W_emb, b_emb):
    n, c = x.shape
    k = W1.shape[0]
    e = kernel_idx.shape[0]
    emb_d = emb.shape[1]
    src = edge_index[0]
    dst = edge_index[1]
    W1r = jnp.transpose(W1, (1, 0, 2)).reshape(c, k * c).astype(jnp.bfloat16)
    W2r = jnp.transpose(W2, (1, 0, 2)).reshape(c, k * c).astype(jnp.bfloat16)
    W_emb_b = W_emb.astype(jnp.bfloat16)

    BN = 200
    G = n // BN
    BE = e // G
    f32 = jnp.float32

    z1, gidx3 = pl.pallas_call(
        functools.partial(_pre_body, k=k),
        grid=(G,),
        in_specs=[
            pl.BlockSpec((BN, c), lambda i: (i, 0)),
            pl.BlockSpec((1, c), lambda i: (0, 0)),
            pl.BlockSpec((1, c), lambda i: (0, 0)),
            pl.BlockSpec((c, k * c), lambda i: (0, 0)),
            pl.BlockSpec((1, 1, BE), lambda i: (i, 0, 0)),
            pl.BlockSpec((1, 1, BE), lambda i: (i, 0, 0)),
        ],
        out_specs=[
            pl.BlockSpec((BN, k * c), lambda i: (i, 0)),
            pl.BlockSpec((1, 1, BE), lambda i: (i, 0, 0)),
        ],
        out_shape=[
            jax.ShapeDtypeStruct((n, k * c), f32),
            jax.ShapeDtypeStruct((G, 1, BE), jnp.int32),
        ],
    )(x, gamma1.reshape(1, c), beta1.reshape(1, c), W1r,
      src.reshape(G, 1, BE), kernel_idx.reshape(G, 1, BE))

    gidx = gidx3.reshape(e)
    y1 = _seg_sum(z1.reshape(n * k, c), gidx, dst, n, c, e)

    eo = pl.pallas_call(
        _emb_body,
        grid=(G,),
        in_specs=[
            pl.BlockSpec((BN, emb_d), lambda i: (i, 0)),
            pl.BlockSpec((emb_d, 2 * c), lambda i: (0, 0)),
            pl.BlockSpec((1, 2 * c), lambda i: (0, 0)),
        ],
        out_specs=pl.BlockSpec((BN, 2 * c), lambda i: (i, 0)),
        out_shape=jax.ShapeDtypeStruct((n, 2 * c), f32),
    )(emb, W_emb_b, b_emb.reshape(1, 2 * c))

    z2 = pl.pallas_call(
        _mid_body,
        grid=(G,),
        in_specs=[
            pl.BlockSpec((2, BN, c), lambda i: (0, i, 0)),
            pl.BlockSpec((1, c), lambda i: (0, 0)),
            pl.BlockSpec((BN, 2 * c), lambda i: (i, 0)),
            pl.BlockSpec((c, k * c), lambda i: (0, 0)),
        ],
        out_specs=pl.BlockSpec((BN, k * c), lambda i: (i, 0)),
        out_shape=jax.ShapeDtypeStruct((n, k * c), f32),
    )(y1, b1.reshape(1, c), eo, W2r)

    y2 = _seg_sum(z2.reshape(n * k, c), gidx, dst, n, c, e)

    out = pl.pallas_call(
        _post_body,
        grid=(G,),
        in_specs=[
            pl.BlockSpec((2, BN, c), lambda i: (0, i, 0)),
            pl.BlockSpec((1, c), lambda i: (0, 0)),
            pl.BlockSpec((BN, c), lambda i: (i, 0)),
        ],
        out_specs=pl.BlockSpec((BN, c), lambda i: (i, 0)),
        out_shape=jax.ShapeDtypeStruct((n, c), f32),
    )(y2, b2.reshape(1, c), x)

    return out


# speedup vs baseline: 8.6654x; 1.5080x over previous
import functools

import jax
import jax.numpy as jnp
from jax import lax
from jax.experimental import pallas as pl
from jax.experimental.pallas import tpu as pltpu
from jax.experimental.pallas import tpu_sc as plsc

_EPS = 1e-6


def _pre_body(x_ref, g_ref, b_ref, w_ref, src_ref, kidx_ref, z_ref, gidx_ref,
              *, k, n):
    c = x_ref.shape[-1]
    h = x_ref[...]
    mu = jnp.mean(h, axis=1, keepdims=True)
    d = h - mu
    var = jnp.mean(d * d, axis=1, keepdims=True)
    hn = d * lax.rsqrt(var + _EPS)
    hn = hn * g_ref[...] + b_ref[...]
    hs = hn * jax.nn.sigmoid(hn)
    big = jnp.dot(hs.astype(jnp.bfloat16), w_ref[...],
                  preferred_element_type=jnp.float32)
    for kk in range(k):
        z_ref[kk] = big[:, kk * c:(kk + 1) * c]
    gidx_ref[...] = kidx_ref[...] * n + src_ref[...]


def _emb_body(emb_ref, we_ref, be_ref, eo_ref):
    e = emb_ref[...]
    e = e * jax.nn.sigmoid(e)
    eo_ref[...] = jnp.dot(e.astype(jnp.bfloat16), we_ref[...],
                          preferred_element_type=jnp.float32) + be_ref[...]


def _mid_body(y_ref, b1_ref, eo_ref, w2_ref, z_ref):
    c = b1_ref.shape[-1]
    h1 = y_ref[0] + y_ref[1] + b1_ref[...]
    mu = jnp.mean(h1, axis=1, keepdims=True)
    d = h1 - mu
    var = jnp.mean(d * d, axis=1, keepdims=True)
    hn = d * lax.rsqrt(var + _EPS)
    eo = eo_ref[...]
    scale = eo[:, :c]
    shift = eo[:, c:]
    h = hn * (1.0 + scale) + shift
    h = h * jax.nn.sigmoid(h)
    big = jnp.dot(h.astype(jnp.bfloat16), w2_ref[...],
                  preferred_element_type=jnp.float32)
    for kk in range(w2_ref.shape[-1] // c):
        z_ref[kk] = big[:, kk * c:(kk + 1) * c]


def _post_body(y_ref, b2_ref, x_ref, o_ref):
    o_ref[...] = y_ref[0] + y_ref[1] + b2_ref[...] + x_ref[...]


@functools.lru_cache(maxsize=None)
def _make_seg(n, c, e):
    NC, NS = 2, 16
    CH = 80
    EPW = e // (NC * NS)
    NITER = EPW // CH
    ZB = 128
    NP = -(-n // (NS * ZB)) * (NS * ZB)
    RPT = NP // NS
    assert EPW * NC * NS == e and NITER * CH == EPW and (RPT // ZB) * ZB == RPT
    assert NITER % 2 == 1

    mesh = plsc.VectorSubcoreMesh(
        core_axis_name="c", subcore_axis_name="s", num_cores=NC, num_subcores=NS)

    @functools.partial(
        pl.kernel,
        out_type=jax.ShapeDtypeStruct((NC, NP, c), jnp.float32),
        mesh=mesh,
        scratch_types=[
            pltpu.VMEM((ZB, c), jnp.float32),
            [pltpu.VMEM((CH, c), jnp.float32)] * 2,
            [pltpu.VMEM((CH,), jnp.int32)] * 2,
            [pltpu.VMEM((CH,), jnp.int32)] * 2,
            pltpu.VMEM_SHARED((NP, c), jnp.float32),
            [pltpu.SemaphoreType.DMA] * 6,
        ],
    )
    def seg(gidx_hbm, dst_hbm, z_hbm, out_hbm, zbuf, rows, gib, dib, yacc,
            sems):
        sg = sems[0:2]
        sn = sems[2:4]
        sd = sems[4:6]
        cc = lax.axis_index("c")
        ss = lax.axis_index("s")
        t = cc * NS + ss
        ebase = t * EPW

        def i_issue(i, b):
            off = pl.multiple_of(ebase + i * CH, 8)
            pltpu.async_copy(gidx_hbm.at[pl.ds(off, CH)], gib[b], sn[b])
            pltpu.async_copy(dst_hbm.at[pl.ds(off, CH)], dib[b], sd[b])

        def i_wait(i, b):
            off = pl.multiple_of(ebase + i * CH, 8)
            pltpu.make_async_copy(gidx_hbm.at[pl.ds(off, CH)], gib[b],
                                  sn[b]).wait()
            pltpu.make_async_copy(dst_hbm.at[pl.ds(off, CH)], dib[b],
                                  sd[b]).wait()

        def g_issue(b):
            pltpu.async_copy(z_hbm.at[gib[b]], rows[b], sg[b])

        def g_wait(b):
            pltpu.make_async_copy(z_hbm.at[gib[b]], rows[b], sg[b]).wait()

        def s_do(b):
            pltpu.sync_copy(rows[b], yacc.at[dib[b]], add=True)

        i_issue(0, 0)
        i_issue(1, 1)

        zero16 = jnp.zeros((16,), jnp.float32)

        def zb_body(i, carry):
            for j in range(c // 16):
                zbuf[i, pl.ds(j * 16, 16)] = zero16
            return carry
        lax.fori_loop(0, ZB, zb_body, 0)

        row0 = ss * RPT
        for r in range(RPT // ZB):
            pltpu.sync_copy(zbuf, yacc.at[pl.ds(row0 + r * ZB, ZB)])
        plsc.subcore_barrier()

        i_wait(0, 0)
        g_issue(0)

        def one(i, b):
            bo = 1 - b
            i_wait(i + 1, bo)
            g_issue(bo)
            g_wait(b)
            s_do(b)
            inx = jnp.minimum(i + 2, NITER - 1)
            i_issue(inx, b)

        def body2(j, carry):
            i0 = j * 2
            one(i0, 0)
            one(i0 + 1, 1)
            return carry
        lax.fori_loop(0, NITER // 2, body2, 0)

        g_wait(0)
        s_do(0)
        i_wait(NITER - 1, 1)

        plsc.subcore_barrier()
        pltpu.sync_copy(yacc.at[pl.ds(row0, RPT)],
                        out_hbm.at[cc, pl.ds(row0, RPT)])

    return seg


def _seg_sum(z2d, gidx, dst, n, c, e):
    seg = _make_seg(n, c, e)
    return seg(gidx, dst, z2d)


def kernel(x, edge_index, kernel_idx, emb, num_frames, gamma1, beta1,
           W1, b1, W2, b2, W_emb, b_emb):
    n, c = x.shape
    k = W1.shape[0]
    e = kernel_idx.shape[0]
    emb_d = emb.shape[1]
    src = edge_index[0]
    dst = edge_index[1]
    W1r = jnp.transpose(W1, (1, 0, 2)).reshape(c, k * c).astype(jnp.bfloat16)
    W2r = jnp.transpose(W2, (1, 0, 2)).reshape(c, k * c).astype(jnp.bfloat16)
    W_emb_b = W_emb.astype(jnp.bfloat16)

    BN = 200
    G = n // BN
    BE = e // G
    f32 = jnp.float32

    z1, gidx3 = pl.pallas_call(
        functools.partial(_pre_body, k=k, n=n),
        grid=(G,),
        in_specs=[
            pl.BlockSpec((BN, c), lambda i: (i, 0)),
            pl.BlockSpec((1, c), lambda i: (0, 0)),
            pl.BlockSpec((1, c), lambda i: (0, 0)),
            pl.BlockSpec((c, k * c), lambda i: (0, 0)),
            pl.BlockSpec((1, 1, BE), lambda i: (i, 0, 0)),
            pl.BlockSpec((1, 1, BE), lambda i: (i, 0, 0)),
        ],
        out_specs=[
            pl.BlockSpec((k, BN, c), lambda i: (0, i, 0)),
            pl.BlockSpec((1, 1, BE), lambda i: (i, 0, 0)),
        ],
        out_shape=[
            jax.ShapeDtypeStruct((k, n, c), f32),
            jax.ShapeDtypeStruct((G, 1, BE), jnp.int32),
        ],
    )(x, gamma1.reshape(1, c), beta1.reshape(1, c), W1r,
      src.reshape(G, 1, BE), kernel_idx.reshape(G, 1, BE))

    gidx = gidx3.reshape(e)
    y1 = _seg_sum(z1.reshape(k * n, c), gidx, dst, n, c, e)

    eo = pl.pallas_call(
        _emb_body,
        grid=(G,),
        in_specs=[
            pl.BlockSpec((BN, emb_d), lambda i: (i, 0)),
            pl.BlockSpec((emb_d, 2 * c), lambda i: (0, 0)),
            pl.BlockSpec((1, 2 * c), lambda i: (0, 0)),
        ],
        out_specs=pl.BlockSpec((BN, 2 * c), lambda i: (i, 0)),
        out_shape=jax.ShapeDtypeStruct((n, 2 * c), f32),
    )(emb, W_emb_b, b_emb.reshape(1, 2 * c))

    z2 = pl.pallas_call(
        _mid_body,
        grid=(G,),
        in_specs=[
            pl.BlockSpec((2, BN, c), lambda i: (0, i, 0)),
            pl.BlockSpec((1, c), lambda i: (0, 0)),
            pl.BlockSpec((BN, 2 * c), lambda i: (i, 0)),
            pl.BlockSpec((c, k * c), lambda i: (0, 0)),
        ],
        out_specs=pl.BlockSpec((k, BN, c), lambda i: (0, i, 0)),
        out_shape=jax.ShapeDtypeStruct((k, n, c), f32),
    )(y1, b1.reshape(1, c), eo, W2r)

    y2 = _seg_sum(z2.reshape(k * n, c), gidx, dst, n, c, e)

    out = pl.pallas_call(
        _post_body,
        grid=(G,),
        in_specs=[
            pl.BlockSpec((2, BN, c), lambda i: (0, i, 0)),
            pl.BlockSpec((1, c), lambda i: (0, 0)),
            pl.BlockSpec((BN, c), lambda i: (i, 0)),
        ],
        out_specs=pl.BlockSpec((BN, c), lambda i: (i, 0)),
        out_shape=jax.ShapeDtypeStruct((n, c), f32),
    )(y2, b2.reshape(1, c), x)

    return out


# speedup vs baseline: 9.7500x; 1.1252x over previous
import functools

import jax
import jax.numpy as jnp
from jax import lax
from jax.experimental import pallas as pl
from jax.experimental.pallas import tpu as pltpu
from jax.experimental.pallas import tpu_sc as plsc

_EPS = 1e-6


def _pre_body(x_ref, g_ref, b_ref, w_ref, src_ref, kidx_ref, z_ref, gidx_ref,
              *, k, n):
    c = x_ref.shape[-1]
    h = x_ref[...]
    mu = jnp.mean(h, axis=1, keepdims=True)
    d = h - mu
    var = jnp.mean(d * d, axis=1, keepdims=True)
    hn = d * lax.rsqrt(var + _EPS)
    hn = hn * g_ref[...] + b_ref[...]
    hs = hn * jax.nn.sigmoid(hn)
    big = jnp.dot(hs.astype(jnp.bfloat16), w_ref[...],
                  preferred_element_type=jnp.float32)
    for kk in range(k):
        z_ref[kk] = big[:, kk * c:(kk + 1) * c]
    gidx_ref[...] = kidx_ref[...] * n + src_ref[...]


def _emb_body(emb_ref, we_ref, be_ref, eo_ref):
    e = emb_ref[...]
    e = e * jax.nn.sigmoid(e)
    eo_ref[...] = jnp.dot(e.astype(jnp.bfloat16), we_ref[...],
                          preferred_element_type=jnp.float32) + be_ref[...]


def _mid_body(y_ref, b1_ref, eo_ref, w2_ref, z_ref):
    c = b1_ref.shape[-1]
    h1 = y_ref[0] + y_ref[1] + b1_ref[...]
    mu = jnp.mean(h1, axis=1, keepdims=True)
    d = h1 - mu
    var = jnp.mean(d * d, axis=1, keepdims=True)
    hn = d * lax.rsqrt(var + _EPS)
    eo = eo_ref[...]
    scale = eo[:, :c]
    shift = eo[:, c:]
    h = hn * (1.0 + scale) + shift
    h = h * jax.nn.sigmoid(h)
    big = jnp.dot(h.astype(jnp.bfloat16), w2_ref[...],
                  preferred_element_type=jnp.float32)
    for kk in range(w2_ref.shape[-1] // c):
        z_ref[kk] = big[:, kk * c:(kk + 1) * c]


def _post_body(y_ref, b2_ref, x_ref, o_ref):
    o_ref[...] = y_ref[0] + y_ref[1] + b2_ref[...] + x_ref[...]


@functools.lru_cache(maxsize=None)
def _make_seg(n, c, e):
    NC, NS = 2, 16
    CH = 80
    EPW = e // (NC * NS)
    NITER = EPW // CH
    ZB = 128
    NP = -(-n // (NS * ZB)) * (NS * ZB)
    RPT = NP // NS
    assert EPW * NC * NS == e and NITER * CH == EPW and (RPT // ZB) * ZB == RPT
    assert NITER % 2 == 1

    mesh = plsc.VectorSubcoreMesh(
        core_axis_name="c", subcore_axis_name="s", num_cores=NC, num_subcores=NS)

    @functools.partial(
        pl.kernel,
        out_type=jax.ShapeDtypeStruct((NC, NP, c), jnp.float32),
        mesh=mesh,
        scratch_types=[
            pltpu.VMEM((ZB, c), jnp.float32),
            [pltpu.VMEM((CH, c), jnp.float32)] * 2,
            [pltpu.VMEM((CH,), jnp.int32)] * 4,
            [pltpu.VMEM((CH,), jnp.int32)] * 4,
            pltpu.VMEM_SHARED((NP, c), jnp.float32),
            [pltpu.SemaphoreType.DMA] * 2,
            [pltpu.SemaphoreType.DMA] * 4,
            [pltpu.SemaphoreType.DMA] * 4,
            [pltpu.SemaphoreType.DMA] * 2,
        ],
    )
    def seg(gidx_hbm, dst_hbm, z_hbm, out_hbm, zbuf, rows, gib, dib, yacc,
            sg, sn, sd, sc):
        cc = lax.axis_index("c")
        ss = lax.axis_index("s")
        t = cc * NS + ss
        ebase = t * EPW

        def i_issue(i, q):
            off = pl.multiple_of(ebase + i * CH, 8)
            pltpu.async_copy(gidx_hbm.at[pl.ds(off, CH)], gib[q], sn[q])
            pltpu.async_copy(dst_hbm.at[pl.ds(off, CH)], dib[q], sd[q])

        def i_wait(i, q):
            off = pl.multiple_of(ebase + i * CH, 8)
            pltpu.make_async_copy(gidx_hbm.at[pl.ds(off, CH)], gib[q],
                                  sn[q]).wait()
            pltpu.make_async_copy(dst_hbm.at[pl.ds(off, CH)], dib[q],
                                  sd[q]).wait()

        def g_issue(b, q):
            pltpu.async_copy(z_hbm.at[gib[q]], rows[b], sg[b])

        def g_wait(b, q):
            pltpu.make_async_copy(z_hbm.at[gib[q]], rows[b], sg[b]).wait()

        def s_issue(b, q):
            pltpu.async_copy(rows[b], yacc.at[dib[q]], sc[b], add=True)

        def s_wait(b, q):
            pltpu.make_async_copy(rows[b], yacc.at[dib[q]], sc[b]).wait()

        i_issue(0, 0)
        i_issue(1, 1)
        i_issue(2, 2)

        zero16 = jnp.zeros((16,), jnp.float32)

        def zb_body(i, carry):
            for j in range(c // 16):
                zbuf[i, pl.ds(j * 16, 16)] = zero16
            return carry
        lax.fori_loop(0, ZB, zb_body, 0)

        row0 = ss * RPT
        for r in range(RPT // ZB):
            pltpu.sync_copy(zbuf, yacc.at[pl.ds(row0 + r * ZB, ZB)])
        plsc.subcore_barrier()

        i_wait(0, 0)
        g_issue(0, 0)
        i_wait(1, 1)
        g_issue(1, 1)
        g_wait(0, 0)
        s_issue(0, 0)
        i_issue(3, 3)

        def one(i, p):
            b = (1 + p) % 2
            q = (1 + p) % 4
            bo = 1 - b
            qn = (q + 1) % 4
            qf = (q + 3) % 4
            i_wait(i + 1, qn)
            s_wait(bo, qf)
            g_issue(bo, qn)
            g_wait(b, q)
            s_issue(b, q)
            i_issue(i + 3, qf)
            return b, q

        def body4(j, carry):
            i0 = 1 + 4 * j
            for p in range(4):
                one(i0 + p, p)
            return carry
        lax.fori_loop(0, (NITER - 5) // 4, body4, 0)

        for p in range(4):
            i = NITER - 4 + p
            b = (1 + p) % 2
            q = (1 + p) % 4
            bo = 1 - b
            qn = (q + 1) % 4
            qf = (q + 3) % 4
            if i + 1 < NITER:
                i_wait(i + 1, qn)
            s_wait(bo, qf)
            if i + 1 < NITER:
                g_issue(bo, qn)
            g_wait(b, q)
            s_issue(b, q)
            if i + 3 < NITER:
                i_issue(i + 3, qf)
        s_wait(0, 0)

        plsc.subcore_barrier()
        pltpu.sync_copy(yacc.at[pl.ds(row0, RPT)],
                        out_hbm.at[cc, pl.ds(row0, RPT)])

    return seg


def _seg_sum(z2d, gidx, dst, n, c, e):
    seg = _make_seg(n, c, e)
    return seg(gidx, dst, z2d)


def kernel(x, edge_index, kernel_idx, emb, num_frames, gamma1, beta1,
           W1, b1, W2, b2, W_emb, b_emb):
    n, c = x.shape
    k = W1.shape[0]
    e = kernel_idx.shape[0]
    emb_d = emb.shape[1]
    src = edge_index[0]
    dst = edge_index[1]
    W1r = jnp.transpose(W1, (1, 0, 2)).reshape(c, k * c).astype(jnp.bfloat16)
    W2r = jnp.transpose(W2, (1, 0, 2)).reshape(c, k * c).astype(jnp.bfloat16)
    W_emb_b = W_emb.astype(jnp.bfloat16)

    BN = 200
    G = n // BN
    BE = e // G
    f32 = jnp.float32

    z1, gidx3 = pl.pallas_call(
        functools.partial(_pre_body, k=k, n=n),
        grid=(G,),
        in_specs=[
            pl.BlockSpec((BN, c), lambda i: (i, 0)),
            pl.BlockSpec((1, c), lambda i: (0, 0)),
            pl.BlockSpec((1, c), lambda i: (0, 0)),
            pl.BlockSpec((c, k * c), lambda i: (0, 0)),
            pl.BlockSpec((1, 1, BE), lambda i: (i, 0, 0)),
            pl.BlockSpec((1, 1, BE), lambda i: (i, 0, 0)),
        ],
        out_specs=[
            pl.BlockSpec((k, BN, c), lambda i: (0, i, 0)),
            pl.BlockSpec((1, 1, BE), lambda i: (i, 0, 0)),
        ],
        out_shape=[
            jax.ShapeDtypeStruct((k, n, c), f32),
            jax.ShapeDtypeStruct((G, 1, BE), jnp.int32),
        ],
    )(x, gamma1.reshape(1, c), beta1.reshape(1, c), W1r,
      src.reshape(G, 1, BE), kernel_idx.reshape(G, 1, BE))

    gidx = gidx3.reshape(e)
    y1 = _seg_sum(z1.reshape(k * n, c), gidx, dst, n, c, e)

    eo = pl.pallas_call(
        _emb_body,
        grid=(G,),
        in_specs=[
            pl.BlockSpec((BN, emb_d), lambda i: (i, 0)),
            pl.BlockSpec((emb_d, 2 * c), lambda i: (0, 0)),
            pl.BlockSpec((1, 2 * c), lambda i: (0, 0)),
        ],
        out_specs=pl.BlockSpec((BN, 2 * c), lambda i: (i, 0)),
        out_shape=jax.ShapeDtypeStruct((n, 2 * c), f32),
    )(emb, W_emb_b, b_emb.reshape(1, 2 * c))

    z2 = pl.pallas_call(
        _mid_body,
        grid=(G,),
        in_specs=[
            pl.BlockSpec((2, BN, c), lambda i: (0, i, 0)),
            pl.BlockSpec((1, c), lambda i: (0, 0)),
            pl.BlockSpec((BN, 2 * c), lambda i: (i, 0)),
            pl.BlockSpec((c, k * c), lambda i: (0, 0)),
        ],
        out_specs=pl.BlockSpec((k, BN, c), lambda i: (0, i, 0)),
        out_shape=jax.ShapeDtypeStruct((k, n, c), f32),
    )(y1, b1.reshape(1, c), eo, W2r)

    y2 = _seg_sum(z2.reshape(k * n, c), gidx, dst, n, c, e)

    out = pl.pallas_call(
        _post_body,
        grid=(G,),
        in_specs=[
            pl.BlockSpec((2, BN, c), lambda i: (0, i, 0)),
            pl.BlockSpec((1, c), lambda i: (0, 0)),
            pl.BlockSpec((BN, c), lambda i: (i, 0)),
        ],
        out_specs=pl.BlockSpec((BN, c), lambda i: (i, 0)),
        out_shape=jax.ShapeDtypeStruct((n, c), f32),
    )(y2, b2.reshape(1, c), x)

    return out


# speedup vs baseline: 10.6024x; 1.0874x over previous
import functools

import jax
import jax.numpy as jnp
from jax import lax
from jax.experimental import pallas as pl
from jax.experimental.pallas import tpu as pltpu
from jax.experimental.pallas import tpu_sc as plsc

_EPS = 1e-6


def _pre_body(x_ref, g_ref, b_ref, w_ref, src_ref, kidx_ref, z_ref, gidx_ref,
              *, k, n):
    c = x_ref.shape[-1]
    h = x_ref[...]
    mu = jnp.mean(h, axis=1, keepdims=True)
    d = h - mu
    var = jnp.mean(d * d, axis=1, keepdims=True)
    hn = d * lax.rsqrt(var + _EPS)
    hn = hn * g_ref[...] + b_ref[...]
    hs = hn * jax.nn.sigmoid(hn)
    big = jnp.dot(hs.astype(jnp.bfloat16), w_ref[...],
                  preferred_element_type=jnp.float32)
    for kk in range(k):
        z_ref[kk] = big[:, kk * c:(kk + 1) * c]
    gidx_ref[...] = kidx_ref[...] * n + src_ref[...]


def _emb_body(emb_ref, we_ref, be_ref, eo_ref):
    e = emb_ref[...]
    e = e * jax.nn.sigmoid(e)
    eo_ref[...] = jnp.dot(e.astype(jnp.bfloat16), we_ref[...],
                          preferred_element_type=jnp.float32) + be_ref[...]


def _mid_body(y_ref, b1_ref, eo_ref, w2_ref, z_ref):
    c = b1_ref.shape[-1]
    h1 = y_ref[0] + y_ref[1] + b1_ref[...]
    mu = jnp.mean(h1, axis=1, keepdims=True)
    d = h1 - mu
    var = jnp.mean(d * d, axis=1, keepdims=True)
    hn = d * lax.rsqrt(var + _EPS)
    eo = eo_ref[...]
    scale = eo[:, :c]
    shift = eo[:, c:]
    h = hn * (1.0 + scale) + shift
    h = h * jax.nn.sigmoid(h)
    big = jnp.dot(h.astype(jnp.bfloat16), w2_ref[...],
                  preferred_element_type=jnp.float32)
    for kk in range(w2_ref.shape[-1] // c):
        z_ref[kk] = big[:, kk * c:(kk + 1) * c]


def _post_body(y_ref, b2_ref, x_ref, o_ref):
    o_ref[...] = y_ref[0] + y_ref[1] + b2_ref[...] + x_ref[...]


@functools.lru_cache(maxsize=None)
def _make_seg(n, c, e):
    NC, NS = 2, 16
    CH = 80
    EPW = e // (NC * NS)
    NITER = EPW // CH
    ZB = 32
    NP = -(-n // (NS * ZB)) * (NS * ZB)
    RPT = NP // NS
    assert EPW * NC * NS == e and NITER * CH == EPW and (RPT // ZB) * ZB == RPT
    assert NITER % 2 == 1

    mesh = plsc.VectorSubcoreMesh(
        core_axis_name="c", subcore_axis_name="s", num_cores=NC, num_subcores=NS)

    @functools.partial(
        pl.kernel,
        out_type=jax.ShapeDtypeStruct((NC, NP, c), jnp.float32),
        mesh=mesh,
        scratch_types=[
            pltpu.VMEM((ZB, c), jnp.float32),
            [pltpu.VMEM((CH, c), jnp.float32)] * 4,
            [pltpu.VMEM((CH,), jnp.int32)] * 4,
            [pltpu.VMEM((CH,), jnp.int32)] * 8,
            pltpu.VMEM_SHARED((NP, c), jnp.float32),
            [pltpu.SemaphoreType.DMA] * 4,
            [pltpu.SemaphoreType.DMA] * 4,
            [pltpu.SemaphoreType.DMA] * 8,
            [pltpu.SemaphoreType.DMA] * 4,
        ],
    )
    def seg(gidx_hbm, dst_hbm, z_hbm, out_hbm, zbuf, rows, gib, dib, yacc,
            sg, sn, sd, sc):
        cc = lax.axis_index("c")
        ss = lax.axis_index("s")
        t = cc * NS + ss
        ebase = t * EPW

        def i_issue(i, q4, q8):
            off = pl.multiple_of(ebase + i * CH, 8)
            pltpu.async_copy(gidx_hbm.at[pl.ds(off, CH)], gib[q4], sn[q4])
            pltpu.async_copy(dst_hbm.at[pl.ds(off, CH)], dib[q8], sd[q8])

        def i_wait(i, q4, q8):
            off = pl.multiple_of(ebase + i * CH, 8)
            pltpu.make_async_copy(gidx_hbm.at[pl.ds(off, CH)], gib[q4],
                                  sn[q4]).wait()
            pltpu.make_async_copy(dst_hbm.at[pl.ds(off, CH)], dib[q8],
                                  sd[q8]).wait()

        def g_issue(b):
            pltpu.async_copy(z_hbm.at[gib[b]], rows[b], sg[b])

        def g_wait(b):
            pltpu.make_async_copy(z_hbm.at[gib[b]], rows[b], sg[b]).wait()

        def s_issue(b, q8):
            pltpu.async_copy(rows[b], yacc.at[dib[q8]], sc[b], add=True)

        def s_wait(b, q8):
            pltpu.make_async_copy(rows[b], yacc.at[dib[q8]], sc[b]).wait()

        i_issue(0, 0, 0)
        i_issue(1, 1, 1)
        i_issue(2, 2, 2)

        zero16 = jnp.zeros((16,), jnp.float32)

        def zb_body(i, carry):
            for j in range(c // 16):
                zbuf[i, pl.ds(j * 16, 16)] = zero16
            return carry
        lax.fori_loop(0, ZB, zb_body, 0)

        row0 = ss * RPT
        for r in range(RPT // ZB):
            pltpu.sync_copy(zbuf, yacc.at[pl.ds(row0 + r * ZB, ZB)])
        plsc.subcore_barrier()

        i_wait(0, 0, 0)
        g_issue(0)
        for i_h in range(3):
            i_wait(i_h + 1, (i_h + 1) % 4, (i_h + 1) % 8)
            g_issue((i_h + 1) % 4)
            g_wait(i_h % 4)
            s_issue(i_h % 4, i_h % 8)
            i_issue(i_h + 3, (i_h + 3) % 4, (i_h + 3) % 8)

        def one(i, b, q8):
            bn = (b + 1) % 4
            i_wait(i + 1, bn, (q8 + 1) % 8)
            s_wait((b + 1) % 4, (q8 + 5) % 8)
            g_issue(bn)
            g_wait(b)
            s_issue(b, q8)
            i_issue(i + 3, (b + 3) % 4, (q8 + 3) % 8)

        def body8(j, carry):
            i0 = 3 + 8 * j
            for p in range(8):
                one(i0 + p, (3 + p) % 4, (3 + p) % 8)
            return carry
        lax.fori_loop(0, (NITER - 11) // 8, body8, 0)

        for p in range(10):
            i = NITER - 10 + p
            b = i % 4
            q8 = i % 8
            bn = (b + 1) % 4
            if i + 1 < NITER:
                i_wait(i + 1, bn, (q8 + 1) % 8)
            s_wait((b + 1) % 4, (q8 + 5) % 8)
            if i + 1 < NITER:
                g_issue(bn)
            g_wait(b)
            s_issue(b, q8)
            if i + 3 < NITER:
                i_issue(i + 3, (b + 3) % 4, (q8 + 3) % 8)
        for i_d in range(NITER - 3, NITER):
            s_wait(i_d % 4, i_d % 8)

        plsc.subcore_barrier()
        pltpu.sync_copy(yacc.at[pl.ds(row0, RPT)],
                        out_hbm.at[cc, pl.ds(row0, RPT)])

    return seg


def _seg_sum(z2d, gidx, dst, n, c, e):
    seg = _make_seg(n, c, e)
    return seg(gidx, dst, z2d)


def kernel(x, edge_index, kernel_idx, emb, num_frames, gamma1, beta1,
           W1, b1, W2, b2, W_emb, b_emb):
    n, c = x.shape
    k = W1.shape[0]
    e = kernel_idx.shape[0]
    emb_d = emb.shape[1]
    src = edge_index[0]
    dst = edge_index[1]
    W1r = jnp.transpose(W1, (1, 0, 2)).reshape(c, k * c).astype(jnp.bfloat16)
    W2r = jnp.transpose(W2, (1, 0, 2)).reshape(c, k * c).astype(jnp.bfloat16)
    W_emb_b = W_emb.astype(jnp.bfloat16)

    BN = 200
    G = n // BN
    BE = e // G
    f32 = jnp.float32

    z1, gidx3 = pl.pallas_call(
        functools.partial(_pre_body, k=k, n=n),
        grid=(G,),
        in_specs=[
            pl.BlockSpec((BN, c), lambda i: (i, 0)),
            pl.BlockSpec((1, c), lambda i: (0, 0)),
            pl.BlockSpec((1, c), lambda i: (0, 0)),
            pl.BlockSpec((c, k * c), lambda i: (0, 0)),
            pl.BlockSpec((1, 1, BE), lambda i: (i, 0, 0)),
            pl.BlockSpec((1, 1, BE), lambda i: (i, 0, 0)),
        ],
        out_specs=[
            pl.BlockSpec((k, BN, c), lambda i: (0, i, 0)),
            pl.BlockSpec((1, 1, BE), lambda i: (i, 0, 0)),
        ],
        out_shape=[
            jax.ShapeDtypeStruct((k, n, c), f32),
            jax.ShapeDtypeStruct((G, 1, BE), jnp.int32),
        ],
    )(x, gamma1.reshape(1, c), beta1.reshape(1, c), W1r,
      src.reshape(G, 1, BE), kernel_idx.reshape(G, 1, BE))

    gidx = gidx3.reshape(e)
    y1 = _seg_sum(z1.reshape(k * n, c), gidx, dst, n, c, e)

    eo = pl.pallas_call(
        _emb_body,
        grid=(G,),
        in_specs=[
            pl.BlockSpec((BN, emb_d), lambda i: (i, 0)),
            pl.BlockSpec((emb_d, 2 * c), lambda i: (0, 0)),
            pl.BlockSpec((1, 2 * c), lambda i: (0, 0)),
        ],
        out_specs=pl.BlockSpec((BN, 2 * c), lambda i: (i, 0)),
        out_shape=jax.ShapeDtypeStruct((n, 2 * c), f32),
    )(emb, W_emb_b, b_emb.reshape(1, 2 * c))

    z2 = pl.pallas_call(
        _mid_body,
        grid=(G,),
        in_specs=[
            pl.BlockSpec((2, BN, c), lambda i: (0, i, 0)),
            pl.BlockSpec((1, c), lambda i: (0, 0)),
            pl.BlockSpec((BN, 2 * c), lambda i: (i, 0)),
            pl.BlockSpec((c, k * c), lambda i: (0, 0)),
        ],
        out_specs=pl.BlockSpec((k, BN, c), lambda i: (0, i, 0)),
        out_shape=jax.ShapeDtypeStruct((k, n, c), f32),
    )(y1, b1.reshape(1, c), eo, W2r)

    y2 = _seg_sum(z2.reshape(k * n, c), gidx, dst, n, c, e)

    out = pl.pallas_call(
        _post_body,
        grid=(G,),
        in_specs=[
            pl.BlockSpec((2, BN, c), lambda i: (0, i, 0)),
            pl.BlockSpec((1, c), lambda i: (0, 0)),
            pl.BlockSpec((BN, c), lambda i: (i, 0)),
        ],
        out_specs=pl.BlockSpec((BN, c), lambda i: (i, 0)),
        out_shape=jax.ShapeDtypeStruct((n, c), f32),
    )(y2, b2.reshape(1, c), x)

    return out
